# baseline (speedup 1.0000x reference)
import jax
import jax.numpy as jnp
from jax.experimental import pallas as pl
import jax.experimental.pallas.tpu as pltpu

HIDDEN = 1024
E = 8
T_TOTAL = 8192
TB = 2048
NUM_TB = T_TOTAL // TB
EPG = E // NUM_TB


def _dot_t(a, b):
    return jax.lax.dot_general(
        a, b, (((1,), (1,)), ((), ())), preferred_element_type=jnp.float32
    )


def _moe_kernel(x_ref, x8_ref, gw_ref, gb_ref, gbt_ref, ew_ref, eb_ref,
                out_ref, logits_ref, c_ref, acc_ref):
    t = pl.program_id(0)

    xb = x_ref[:, :]
    gw = gw_ref[:, :]
    logits_ref[:, :] = _dot_t(gw, xb) + gbt_ref[:, :]

    out_ref[:, :] = jnp.zeros((TB, HIDDEN), jnp.float32)

    x8 = x8_ref[:, :]

    @pl.when(t == 0)
    def _init():
        l8 = _dot_t(x8, gw) + gb_ref[:, :]
        m = jnp.max(l8, axis=-1, keepdims=True)
        p = jnp.exp(l8 - m)
        p = p / jnp.sum(p, axis=-1, keepdims=True)

        iota_e = jax.lax.broadcasted_iota(
            jnp.int32, (E, E), 1).astype(jnp.float32)
        iota_r = jax.lax.broadcasted_iota(
            jnp.int32, (E, E), 0).astype(jnp.float32)

        rank = jnp.zeros((E, E), jnp.float32)
        for e2 in range(E):
            pe2 = p[:, e2:e2 + 1]
            rank = rank + jnp.where(
                (pe2 > p) | ((pe2 == p) & (e2 < iota_e)), 1.0, 0.0)

        sp = jnp.zeros((E, E), jnp.float32)
        for e in range(E):
            sp = sp + jnp.where(rank[:, e:e + 1] == iota_e,
                                p[:, e:e + 1], 0.0)
        sp = sp / jnp.sum(sp, axis=-1, keepdims=True)

        for i in range(E):
            ri = rank[i:i + 1, :]
            c_ref[i] = sp * jnp.where(ri == iota_r, 1.0, 0.0)

        acc_ref[:, :] = jnp.zeros((E, HIDDEN), jnp.float32)

    upd = jnp.zeros((E, HIDDEN), jnp.float32)
    for j in range(EPG):
        y = _dot_t(x8, ew_ref[j]) + eb_ref[j]
        upd = upd + jax.lax.dot_general(
            c_ref[t * EPG + j], y, (((1,), (0,)), ((), ())),
            preferred_element_type=jnp.float32)
    acc_ref[:, :] += upd

    @pl.when(t == NUM_TB - 1)
    def _final():
        out_ref[0:E, :] = acc_ref[:, :]


@jax.jit
def kernel(x, gate_W, gate_b, expert_W, expert_b):
    B, S, H = x.shape
    xf = x.reshape(B * S, H)
    gb2 = gate_b.reshape(1, E)
    gbt = gate_b.reshape(E, 1)
    ew_half = expert_W
    eb_half = expert_b.reshape(E, 1, H)

    out, logits_t = pl.pallas_call(
        _moe_kernel,
        grid=(NUM_TB,),
        in_specs=[
            pl.BlockSpec((TB, H), lambda i: ((i + 1) % NUM_TB, 0)),
            pl.BlockSpec((E, H), lambda i: (0, 0)),
            pl.BlockSpec((E, H), lambda i: (0, 0)),
            pl.BlockSpec((1, E), lambda i: (0, 0)),
            pl.BlockSpec((E, 1), lambda i: (0, 0)),
            pl.BlockSpec((EPG, H, H), lambda i: (i, 0, 0)),
            pl.BlockSpec((EPG, 1, H), lambda i: (i, 0, 0)),
        ],
        out_specs=[
            pl.BlockSpec((TB, H), lambda i: ((i + 1) % NUM_TB, 0)),
            pl.BlockSpec((E, TB), lambda i: (0, (i + 1) % NUM_TB)),
        ],
        out_shape=[
            jax.ShapeDtypeStruct((B * S, H), jnp.float32),
            jax.ShapeDtypeStruct((E, B * S), jnp.float32),
        ],
        scratch_shapes=[
            pltpu.VMEM((E, E, E), jnp.float32),
            pltpu.VMEM((E, HIDDEN), jnp.float32),
        ],
    )(xf, xf, gate_W, gb2, gbt, ew_half, eb_half)

    return out.reshape(B, S, H), logits_t.T


# speedup vs baseline: 1.0378x; 1.0378x over previous
import jax
import jax.numpy as jnp
from jax.experimental import pallas as pl
import jax.experimental.pallas.tpu as pltpu

HIDDEN = 1024
E = 8
T_TOTAL = 8192
TB = 1024
NUM_TB = T_TOTAL // TB


def _dot_t(a, b):
    return jax.lax.dot_general(
        a, b, (((1,), (1,)), ((), ())), preferred_element_type=jnp.float32
    )


def _moe_kernel(x_ref, x8_ref, gw_ref, gb_ref, gbt_ref, ewa_ref, ewb_ref,
                eb_ref, out_ref, logits_ref, c_ref, acc_ref):
    t = pl.program_id(0)

    xb = x_ref[:, :]
    gw = gw_ref[:, :]
    logits_ref[:, :] = _dot_t(gw, xb) + gbt_ref[:, :]

    out_ref[:, :] = jnp.zeros((TB, HIDDEN), jnp.float32)

    x8 = x8_ref[:, :]

    @pl.when(t == 0)
    def _init():
        l8 = _dot_t(x8, gw) + gb_ref[:, :]
        m = jnp.max(l8, axis=-1, keepdims=True)
        p = jnp.exp(l8 - m)
        p = p / jnp.sum(p, axis=-1, keepdims=True)

        iota_e = jax.lax.broadcasted_iota(
            jnp.int32, (E, E), 1).astype(jnp.float32)
        iota_r = jax.lax.broadcasted_iota(
            jnp.int32, (E, E), 0).astype(jnp.float32)

        rank = jnp.zeros((E, E), jnp.float32)
        for e2 in range(E):
            pe2 = p[:, e2:e2 + 1]
            rank = rank + jnp.where(
                (pe2 > p) | ((pe2 == p) & (e2 < iota_e)), 1.0, 0.0)

        sp = jnp.zeros((E, E), jnp.float32)
        for e in range(E):
            sp = sp + jnp.where(rank[:, e:e + 1] == iota_e,
                                p[:, e:e + 1], 0.0)
        sp = sp / jnp.sum(sp, axis=-1, keepdims=True)

        for i in range(E):
            ri = rank[i:i + 1, :]
            c_ref[i] = sp * jnp.where(ri == iota_r, 1.0, 0.0)

        acc_ref[:, :, :] = jnp.zeros((2, E, HIDDEN // 2), jnp.float32)

    c = c_ref[t]
    ya = _dot_t(x8, ewa_ref[0]) + eb_ref[0, :, 0:HIDDEN // 2]
    yb = _dot_t(x8, ewb_ref[0]) + eb_ref[0, :, HIDDEN // 2:]

    acc_ref[0] += jax.lax.dot_general(
        c, ya, (((1,), (0,)), ((), ())), preferred_element_type=jnp.float32)
    acc_ref[1] += jax.lax.dot_general(
        c, yb, (((1,), (0,)), ((), ())), preferred_element_type=jnp.float32)

    @pl.when(t == NUM_TB - 1)
    def _final():
        out_ref[0:E, 0:HIDDEN // 2] = acc_ref[0]
        out_ref[0:E, HIDDEN // 2:HIDDEN] = acc_ref[1]


@jax.jit
def kernel(x, gate_W, gate_b, expert_W, expert_b):
    B, S, H = x.shape
    xf = x.reshape(B * S, H)
    gb2 = gate_b.reshape(1, E)
    gbt = gate_b.reshape(E, 1)
    ew_half = expert_W.reshape(2 * E, H // 2, H)
    eb_half = expert_b.reshape(E, 1, H)

    out, logits_t = pl.pallas_call(
        _moe_kernel,
        grid=(NUM_TB,),
        in_specs=[
            pl.BlockSpec((TB, H), lambda i: ((i + 1) % NUM_TB, 0)),
            pl.BlockSpec((E, H), lambda i: (0, 0)),
            pl.BlockSpec((E, H), lambda i: (0, 0)),
            pl.BlockSpec((1, E), lambda i: (0, 0)),
            pl.BlockSpec((E, 1), lambda i: (0, 0)),
            pl.BlockSpec((1, H // 2, H), lambda i: (2 * i, 0, 0)),
            pl.BlockSpec((1, H // 2, H), lambda i: (2 * i + 1, 0, 0)),
            pl.BlockSpec((1, 1, H), lambda i: (i, 0, 0)),
        ],
        out_specs=[
            pl.BlockSpec((TB, H), lambda i: ((i + 1) % NUM_TB, 0)),
            pl.BlockSpec((E, TB), lambda i: (0, (i + 1) % NUM_TB)),
        ],
        out_shape=[
            jax.ShapeDtypeStruct((B * S, H), jnp.float32),
            jax.ShapeDtypeStruct((E, B * S), jnp.float32),
        ],
        scratch_shapes=[
            pltpu.VMEM((E, E, E), jnp.float32),
            pltpu.VMEM((2, E, HIDDEN // 2), jnp.float32),
        ],
    )(xf, xf, gate_W, gb2, gbt, ew_half, ew_half, eb_half)

    return out.reshape(B, S, H), logits_t.T
